# baseline (speedup 1.0000x reference)
import functools

import jax
import jax.numpy as jnp
from jax import lax
from jax.experimental import pallas as pl
from jax.experimental.pallas import tpu as pltpu
from jax.experimental.pallas import tpu_sc as plsc

D_MODEL = 64
D_PAD = 128
NUM_CORES = 2
NUM_SUBCORES = 16
NUM_WORKERS = NUM_CORES * NUM_SUBCORES
NBUF = 2


def _emb_body(idx_hbm, table_hbm, out_hbm, idx_v, stg, gsems, ssems, *,
              rows_per_w, t):
    c = lax.axis_index("c")
    s = lax.axis_index("s")
    wid = s * NUM_CORES + c
    row0 = wid * rows_per_w
    base = row0 * t

    pltpu.sync_copy(idx_hbm.at[pl.ds(row0, rows_per_w)], idx_v)

    def start_gather(g, b):
        pltpu.async_copy(table_hbm.at[idx_v.at[g]], stg[b], gsems[b])

    def start_store(g, b):
        pltpu.async_copy(stg[b],
                         out_hbm.at[pl.ds(base + g * t, t), pl.ds(0, D_MODEL)],
                         ssems[b])

    def wait_gather(b):
        pltpu.make_async_copy(table_hbm.at[pl.ds(0, t)], stg[b],
                              gsems[b]).wait()

    def wait_store(b):
        pltpu.make_async_copy(stg[b],
                              out_hbm.at[pl.ds(0, t), pl.ds(0, D_MODEL)],
                              ssems[b]).wait()

    for b in range(NBUF):
        start_gather(b, b)

    @pl.loop(0, rows_per_w - NBUF, step=NBUF)
    def _(k):
        for b in range(NBUF):
            g = k + b
            wait_gather(b)
            start_store(g, b)
            wait_store(b)
            start_gather(g + NBUF, b)

    for b in range(NBUF):
        wait_gather(b)
        start_store(rows_per_w - NBUF + b, b)
    for b in range(NBUF):
        wait_store(b)


def kernel(input, weight):
    bsz, t = input.shape
    n = bsz * t
    assert bsz % NUM_WORKERS == 0 and (t * D_MODEL * 4) % 64 == 0
    rows_per_w = bsz // NUM_WORKERS
    idx = input.astype(jnp.int32)

    body = functools.partial(_emb_body, rows_per_w=rows_per_w, t=t)
    mesh = plsc.VectorSubcoreMesh(core_axis_name="c", subcore_axis_name="s")
    out = pl.kernel(
        body,
        out_type=jax.ShapeDtypeStruct((n, D_PAD), jnp.float32),
        mesh=mesh,
        compiler_params=pltpu.CompilerParams(
            use_tc_tiling_on_sc=False,
            skip_device_barrier=True,
            disable_bounds_checks=True,
            disable_semaphore_checks=True,
        ),
        scratch_types=[
            pltpu.VMEM((rows_per_w, t), jnp.int32),
            [pltpu.VMEM((t, D_MODEL), jnp.float32) for _ in range(NBUF)],
            [pltpu.SemaphoreType.DMA for _ in range(NBUF)],
            [pltpu.SemaphoreType.DMA for _ in range(NBUF)],
        ],
    )(idx, weight)
    return out[:, :D_MODEL].reshape(bsz, t, D_MODEL)
